# baseline (speedup 1.0000x reference)
import functools

import jax
import jax.numpy as jnp
from jax import lax
from jax.experimental import pallas as pl
from jax.experimental.pallas import tpu as pltpu
import jax.experimental.pallas.tpu_sc as plsc

_N = 10000
_E = 320000
_D = 128
_NCLS = 40
_NC = 2
_NS = 16
_NW = _NC * _NS
_EPW = _E // _NW
_C = 80
_NCHUNK = _EPW // _C
_STR = 640
_NPAD = _NS * _STR

_ZCH = _STR // _C
_LC = 128
_NCH = 80
_EPAD = _NW * _NCH * _LC

_sc_mesh = plsc.VectorSubcoreMesh(core_axis_name="c", subcore_axis_name="s")


_DK = 8


def _deg_body(dst_hbm, out_hbm, dst_all, onesv, zbuf, dacc, sem):
    c = lax.axis_index("c")
    s = lax.axis_index("s")
    wid = s * _NC + c

    def fill_z(j, carry):
        zbuf[pl.ds(j * 16, 16)] = jnp.zeros((16,), jnp.float32)
        return carry

    lax.fori_loop(0, _STR // 16, fill_z, 0)

    def fill_o(j, carry):
        onesv[pl.ds(j * 16, 16)] = jnp.ones((16,), jnp.float32)
        return carry

    lax.fori_loop(0, _LC // 16, fill_o, 0)

    pltpu.sync_copy(zbuf, dacc.at[pl.ds(s * _STR, _STR)])
    pltpu.sync_copy(dst_hbm.at[wid], dst_all)
    plsc.subcore_barrier()

    def body(i, carry):
        for k in range(_DK):
            pltpu.make_async_copy(
                onesv, dacc.at[dst_all.at[i * _DK + k]], sem).start(add=True)
        for k in range(_DK):
            pltpu.make_async_copy(
                onesv, dacc.at[dst_all.at[i * _DK + k]], sem).wait()
        return carry

    lax.fori_loop(0, _NCH // _DK, body, 0)
    plsc.subcore_barrier()

    @pl.when(s == 0)
    def _():
        pltpu.sync_copy(dacc, out_hbm.at[c])


_deg_call = pl.kernel(
    _deg_body,
    out_type=jax.ShapeDtypeStruct((_NC, _NPAD), jnp.float32),
    mesh=_sc_mesh,
    scratch_types=[
        pltpu.VMEM((_NCH, _LC), jnp.int32),
        pltpu.VMEM((_LC,), jnp.float32),
        pltpu.VMEM((_STR,), jnp.float32),
        pltpu.VMEM_SHARED((_NPAD,), jnp.float32),
        pltpu.SemaphoreType.DMA,
    ],
)


def _gather_scatter_body(g_hbm, src_hbm, dst_hbm, out_hbm,
                         bufa, bufb, sxa, sxb, dxa, dxb, acc,
                         gsa, gsb, ssa, ssb, isa, isb, ida, idb):
    c = lax.axis_index("c")
    s = lax.axis_index("s")
    wid = s * _NC + c

    def fz(i, carry):
        def fz2(j, carry2):
            bufa[i, pl.ds(j * 16, 16)] = jnp.zeros((16,), jnp.float32)
            return carry2

        lax.fori_loop(0, _D // 16, fz2, 0)
        return carry

    lax.fori_loop(0, _LC, fz, 0)

    def za(k, carry):
        pltpu.sync_copy(bufa, acc.at[pl.ds(s * _STR + k * _LC, _LC)])
        return carry

    lax.fori_loop(0, _STR // _LC, za, 0)
    plsc.subcore_barrier()

    def g_desc(ch, buf, sx, sem):
        return pltpu.make_async_copy(g_hbm.at[sx], buf, sem)

    def s_desc(ch, buf, dx, sem):
        return pltpu.make_async_copy(buf, acc.at[dx], sem)

    ibase = wid * (_NCH * _LC)

    def sx_desc(ch, sx, sem):
        return pltpu.make_async_copy(src_hbm.at[pl.ds(ibase + ch * _LC, _LC)],
                                     sx, sem)

    def dx_desc(ch, dx, sem):
        return pltpu.make_async_copy(dst_hbm.at[pl.ds(ibase + ch * _LC, _LC)],
                                     dx, sem)

    sx_desc(0, sxa, isa).start()
    sx_desc(0, sxa, isa).wait()
    sx_desc(1, sxb, isb).start()
    sx_desc(1, sxb, isb).wait()
    g_desc(0, bufa, sxa, gsa).start()
    g_desc(1, bufb, sxb, gsb).start()
    dx_desc(0, dxa, ida).start()
    dx_desc(1, dxb, idb).start()

    def land(c0, c2, buf, sx, dx, gs, ss, isem):
        g_desc(c0, buf, sx, gs).wait()

        @pl.when(c2 < _NCH)
        def _():
            sx_desc(c2, sx, isem).start()

        dx_desc(c0, dx, ida if ss is ssa else idb).wait()
        s_desc(c0, buf, dx, ss).start(add=True)

    def refill(c0, c2, buf, sx, dx, gs, ss, isem, idsem):
        s_desc(c0, buf, dx, ss).wait()

        @pl.when(c2 < _NCH)
        def _():
            sx_desc(c2, sx, isem).wait()
            g_desc(c2, buf, sx, gs).start()
            dx_desc(c2, dx, idsem).start()

    def body(j, carry):
        c0, c1, c2, c3 = 2 * j, 2 * j + 1, 2 * j + 2, 2 * j + 3
        land(c0, c2, bufa, sxa, dxa, gsa, ssa, isa)
        land(c1, c3, bufb, sxb, dxb, gsb, ssb, isb)
        refill(c0, c2, bufa, sxa, dxa, gsa, ssa, isa, ida)
        refill(c1, c3, bufb, sxb, dxb, gsb, ssb, isb, idb)
        return carry

    lax.fori_loop(0, _NCH // 2, body, 0)
    plsc.subcore_barrier()

    pltpu.sync_copy(acc.at[pl.ds(s * _STR, _STR)],
                    out_hbm.at[c, pl.ds(s * _STR, _STR)])


_gather_scatter_call = pl.kernel(
    _gather_scatter_body,
    out_type=jax.ShapeDtypeStruct((_NC, _NPAD, _D), jnp.float32),
    mesh=_sc_mesh,
    scratch_types=[
        pltpu.VMEM((_LC, _D), jnp.float32),
        pltpu.VMEM((_LC, _D), jnp.float32),
        pltpu.VMEM((_LC,), jnp.int32),
        pltpu.VMEM((_LC,), jnp.int32),
        pltpu.VMEM((_LC,), jnp.int32),
        pltpu.VMEM((_LC,), jnp.int32),
        pltpu.VMEM_SHARED((_NPAD, _D), jnp.float32),
        pltpu.SemaphoreType.DMA,
        pltpu.SemaphoreType.DMA,
        pltpu.SemaphoreType.DMA,
        pltpu.SemaphoreType.DMA,
        pltpu.SemaphoreType.DMA,
        pltpu.SemaphoreType.DMA,
        pltpu.SemaphoreType.DMA,
        pltpu.SemaphoreType.DMA,
    ],
)


_R = 400
_G = _N // _R


def _tc_first_body(x_ref, w_ref, d0_ref, d1_ref, h_ref, g_ref, di_ref):
    deg = d0_ref[...] + d1_ref[...] + 1.0
    dinv = lax.rsqrt(deg)
    h = jnp.dot(x_ref[...], w_ref[...], preferred_element_type=jnp.float32)
    h_ref[...] = h
    g_ref[...] = h * dinv
    di_ref[...] = dinv


_tc_first = pl.pallas_call(
    _tc_first_body,
    grid=(_G,),
    in_specs=[
        pl.BlockSpec((_R, _D), lambda i: (i, 0)),
        pl.BlockSpec((_D, _D), lambda i: (0, 0)),
        pl.BlockSpec((_R, 1), lambda i: (i, 0)),
        pl.BlockSpec((_R, 1), lambda i: (i, 0)),
    ],
    out_specs=[
        pl.BlockSpec((_R, _D), lambda i: (i, 0)),
        pl.BlockSpec((_R, _D), lambda i: (i, 0)),
        pl.BlockSpec((_R, 1), lambda i: (i, 0)),
    ],
    out_shape=[
        jax.ShapeDtypeStruct((_N, _D), jnp.float32),
        jax.ShapeDtypeStruct((_N, _D), jnp.float32),
        jax.ShapeDtypeStruct((_N, 1), jnp.float32),
    ],
)


def _tc_mid_body(s0_ref, s1_ref, h_ref, di_ref, b_ref, w_ref,
                 h2_ref, g2_ref):
    di = di_ref[...]
    t = di * (s0_ref[...] + s1_ref[...]) + di * di * h_ref[...] + b_ref[...]
    t = jnp.maximum(t, 0.0)
    h2 = jnp.dot(t, w_ref[...], preferred_element_type=jnp.float32)
    h2_ref[...] = h2
    g2_ref[...] = h2 * di


_tc_mid = pl.pallas_call(
    _tc_mid_body,
    grid=(_G,),
    in_specs=[
        pl.BlockSpec((_R, _D), lambda i: (i, 0)),
        pl.BlockSpec((_R, _D), lambda i: (i, 0)),
        pl.BlockSpec((_R, _D), lambda i: (i, 0)),
        pl.BlockSpec((_R, 1), lambda i: (i, 0)),
        pl.BlockSpec((1, _D), lambda i: (0, 0)),
        pl.BlockSpec((_D, _D), lambda i: (0, 0)),
    ],
    out_specs=[
        pl.BlockSpec((_R, _D), lambda i: (i, 0)),
        pl.BlockSpec((_R, _D), lambda i: (i, 0)),
    ],
    out_shape=[
        jax.ShapeDtypeStruct((_N, _D), jnp.float32),
        jax.ShapeDtypeStruct((_N, _D), jnp.float32),
    ],
)


def _tc_last_body(s0_ref, s1_ref, h_ref, di_ref, b_ref, w_ref, b3_ref,
                  out_ref):
    di = di_ref[...]
    t = di * (s0_ref[...] + s1_ref[...]) + di * di * h_ref[...] + b_ref[...]
    t = jnp.maximum(t, 0.0)
    out_ref[...] = (
        jnp.dot(t, w_ref[...], preferred_element_type=jnp.float32)
        + b3_ref[...]
    )


_tc_last = pl.pallas_call(
    _tc_last_body,
    grid=(_G,),
    in_specs=[
        pl.BlockSpec((_R, _D), lambda i: (i, 0)),
        pl.BlockSpec((_R, _D), lambda i: (i, 0)),
        pl.BlockSpec((_R, _D), lambda i: (i, 0)),
        pl.BlockSpec((_R, 1), lambda i: (i, 0)),
        pl.BlockSpec((1, _D), lambda i: (0, 0)),
        pl.BlockSpec((_D, _NCLS), lambda i: (0, 0)),
        pl.BlockSpec((1, _NCLS), lambda i: (0, 0)),
    ],
    out_specs=pl.BlockSpec((_R, _NCLS), lambda i: (i, 0)),
    out_shape=jax.ShapeDtypeStruct((_N, _NCLS), jnp.float32),
)


def kernel(x, edge_index, W1, b1, W2, b2, W3, b3):
    src = edge_index[0]
    dst = edge_index[1]

    pad = _EPAD - _E
    pad_src = (jnp.arange(pad, dtype=src.dtype) * 97) % _N
    pad_dst = _N + (jnp.arange(pad, dtype=dst.dtype) % (_NPAD - _N))
    src3 = jnp.concatenate([src, pad_src])
    dst3 = jnp.concatenate([dst, pad_dst])

    deg_p = _deg_call(dst3.reshape(_NW, _NCH, _LC))
    dp0 = deg_p[0, :_N, None]
    dp1 = deg_p[1, :_N, None]

    h1, g1, dinv = _tc_first(x, W1, dp0, dp1)

    acc1 = _gather_scatter_call(g1, src3, dst3)
    h2, g2 = _tc_mid(acc1[0, :_N], acc1[1, :_N], h1, dinv,
                     b1.reshape(1, _D), W2)

    acc2 = _gather_scatter_call(g2, src3, dst3)
    return _tc_last(acc2[0, :_N], acc2[1, :_N], h2, dinv,
                    b2.reshape(1, _D), W3, b3.reshape(1, _NCLS))


# speedup vs baseline: 1.2807x; 1.2807x over previous
import functools

import jax
import jax.numpy as jnp
from jax import lax
from jax.experimental import pallas as pl
from jax.experimental.pallas import tpu as pltpu
import jax.experimental.pallas.tpu_sc as plsc

_N = 10000
_E = 320000
_D = 128
_NCLS = 40
_NC = 2
_NS = 16
_NW = _NC * _NS
_EPW = _E // _NW
_C = 80
_NCHUNK = _EPW // _C
_STR = 640
_NPAD = _NS * _STR

_ZCH = _STR // _C
_LC = 128
_NCH = 80
_EPAD = _NW * _NCH * _LC

_sc_mesh = plsc.VectorSubcoreMesh(core_axis_name="c", subcore_axis_name="s")


_DK = 8


def _deg_body(dst_hbm, out_hbm, dst_all, onesv, zbuf, dacc, sem):
    c = lax.axis_index("c")
    s = lax.axis_index("s")
    wid = s * _NC + c

    def fill_z(j, carry):
        zbuf[pl.ds(j * 16, 16)] = jnp.zeros((16,), jnp.float32)
        return carry

    lax.fori_loop(0, _STR // 16, fill_z, 0)

    def fill_o(j, carry):
        onesv[pl.ds(j * 16, 16)] = jnp.ones((16,), jnp.float32)
        return carry

    lax.fori_loop(0, _LC // 16, fill_o, 0)

    pltpu.sync_copy(zbuf, dacc.at[pl.ds(s * _STR, _STR)])
    pltpu.sync_copy(dst_hbm.at[wid], dst_all)
    plsc.subcore_barrier()

    def body(i, carry):
        for k in range(_DK):
            pltpu.make_async_copy(
                onesv, dacc.at[dst_all.at[i * _DK + k]], sem).start(add=True)
        for k in range(_DK):
            pltpu.make_async_copy(
                onesv, dacc.at[dst_all.at[i * _DK + k]], sem).wait()
        return carry

    lax.fori_loop(0, _NCH // _DK, body, 0)
    plsc.subcore_barrier()

    @pl.when(s == 0)
    def _():
        pltpu.sync_copy(dacc, out_hbm.at[c])


_deg_call = pl.kernel(
    _deg_body,
    out_type=jax.ShapeDtypeStruct((_NC, _NPAD), jnp.float32),
    mesh=_sc_mesh,
    scratch_types=[
        pltpu.VMEM((_NCH, _LC), jnp.int32),
        pltpu.VMEM((_LC,), jnp.float32),
        pltpu.VMEM((_STR,), jnp.float32),
        pltpu.VMEM_SHARED((_NPAD,), jnp.float32),
        pltpu.SemaphoreType.DMA,
    ],
)


def _gather_scatter_body(g_hbm, src_hbm, dst_hbm, out_hbm,
                         bufa, bufb, sxa, sxb, dxa, dxb, acc,
                         gsa, gsb, ssa, ssb, isa, isb, ida, idb):
    c = lax.axis_index("c")
    s = lax.axis_index("s")
    wid = s * _NC + c

    def fz(i, carry):
        def fz2(j, carry2):
            bufa[i, pl.ds(j * 16, 16)] = jnp.zeros((16,), jnp.float32)
            return carry2

        lax.fori_loop(0, _D // 16, fz2, 0)
        return carry

    lax.fori_loop(0, _LC, fz, 0)

    def za(k, carry):
        pltpu.sync_copy(bufa, acc.at[pl.ds(s * _STR + k * _LC, _LC)])
        return carry

    lax.fori_loop(0, _STR // _LC, za, 0)
    plsc.subcore_barrier()

    def g_desc(ch, buf, sx, sem):
        return pltpu.make_async_copy(g_hbm.at[sx], buf, sem)

    def s_desc(ch, buf, dx, sem):
        return pltpu.make_async_copy(buf, acc.at[dx], sem)

    ibase = wid * (_NCH * _LC)

    def sx_desc(ch, sx, sem):
        return pltpu.make_async_copy(src_hbm.at[pl.ds(ibase + ch * _LC, _LC)],
                                     sx, sem)

    def dx_desc(ch, dx, sem):
        return pltpu.make_async_copy(dst_hbm.at[pl.ds(ibase + ch * _LC, _LC)],
                                     dx, sem)

    sx_desc(0, sxa, isa).start()
    sx_desc(0, sxa, isa).wait()
    sx_desc(1, sxb, isb).start()
    sx_desc(1, sxb, isb).wait()
    g_desc(0, bufa, sxa, gsa).start()
    g_desc(1, bufb, sxb, gsb).start()
    dx_desc(0, dxa, ida).start()
    dx_desc(1, dxb, idb).start()

    def half(c0, c2, buf, sx, dx, gs, ss, isem, idsem):
        g_desc(c0, buf, sx, gs).wait()

        @pl.when(c2 < _NCH)
        def _():
            sx_desc(c2, sx, isem).start()

        dx_desc(c0, dx, idsem).wait()
        s_desc(c0, buf, dx, ss).start(add=True)
        s_desc(c0, buf, dx, ss).wait()

        @pl.when(c2 < _NCH)
        def _():
            sx_desc(c2, sx, isem).wait()
            g_desc(c2, buf, sx, gs).start()
            dx_desc(c2, dx, idsem).start()

    def body(j, carry):
        half(2 * j, 2 * j + 2, bufa, sxa, dxa, gsa, ssa, isa, ida)
        half(2 * j + 1, 2 * j + 3, bufb, sxb, dxb, gsb, ssb, isb, idb)
        return carry

    lax.fori_loop(0, _NCH // 2, body, 0)
    plsc.subcore_barrier()

    pltpu.sync_copy(acc.at[pl.ds(s * _STR, _STR)],
                    out_hbm.at[c, pl.ds(s * _STR, _STR)])


_gather_scatter_call = pl.kernel(
    _gather_scatter_body,
    out_type=jax.ShapeDtypeStruct((_NC, _NPAD, _D), jnp.float32),
    mesh=_sc_mesh,
    scratch_types=[
        pltpu.VMEM((_LC, _D), jnp.float32),
        pltpu.VMEM((_LC, _D), jnp.float32),
        pltpu.VMEM((_LC,), jnp.int32),
        pltpu.VMEM((_LC,), jnp.int32),
        pltpu.VMEM((_LC,), jnp.int32),
        pltpu.VMEM((_LC,), jnp.int32),
        pltpu.VMEM_SHARED((_NPAD, _D), jnp.float32),
        pltpu.SemaphoreType.DMA,
        pltpu.SemaphoreType.DMA,
        pltpu.SemaphoreType.DMA,
        pltpu.SemaphoreType.DMA,
        pltpu.SemaphoreType.DMA,
        pltpu.SemaphoreType.DMA,
        pltpu.SemaphoreType.DMA,
        pltpu.SemaphoreType.DMA,
    ],
)


_R = 400
_G = _N // _R


def _tc_first_body(x_ref, w_ref, d_ref, h_ref, g_ref, di_ref):
    deg = d_ref[0] + d_ref[1] + 1.0
    dinv = lax.rsqrt(deg)
    h = jnp.dot(x_ref[...], w_ref[...], preferred_element_type=jnp.float32)
    h_ref[...] = h
    g_ref[...] = h * dinv
    di_ref[...] = dinv


_tc_first = pl.pallas_call(
    _tc_first_body,
    grid=(_G,),
    in_specs=[
        pl.BlockSpec((_R, _D), lambda i: (i, 0)),
        pl.BlockSpec((_D, _D), lambda i: (0, 0)),
        pl.BlockSpec((_NC, _R, 1), lambda i: (0, i, 0)),
    ],
    out_specs=[
        pl.BlockSpec((_R, _D), lambda i: (i, 0)),
        pl.BlockSpec((_R, _D), lambda i: (i, 0)),
        pl.BlockSpec((_R, 1), lambda i: (i, 0)),
    ],
    out_shape=[
        jax.ShapeDtypeStruct((_N, _D), jnp.float32),
        jax.ShapeDtypeStruct((_N, _D), jnp.float32),
        jax.ShapeDtypeStruct((_N, 1), jnp.float32),
    ],
)


def _tc_mid_body(sp_ref, h_ref, di_ref, b_ref, w_ref,
                 h2_ref, g2_ref):
    di = di_ref[...]
    t = di * (sp_ref[0] + sp_ref[1]) + di * di * h_ref[...] + b_ref[...]
    t = jnp.maximum(t, 0.0)
    h2 = jnp.dot(t, w_ref[...], preferred_element_type=jnp.float32)
    h2_ref[...] = h2
    g2_ref[...] = h2 * di


_tc_mid = pl.pallas_call(
    _tc_mid_body,
    grid=(_G,),
    in_specs=[
        pl.BlockSpec((_NC, _R, _D), lambda i: (0, i, 0)),
        pl.BlockSpec((_R, _D), lambda i: (i, 0)),
        pl.BlockSpec((_R, 1), lambda i: (i, 0)),
        pl.BlockSpec((1, _D), lambda i: (0, 0)),
        pl.BlockSpec((_D, _D), lambda i: (0, 0)),
    ],
    out_specs=[
        pl.BlockSpec((_R, _D), lambda i: (i, 0)),
        pl.BlockSpec((_R, _D), lambda i: (i, 0)),
    ],
    out_shape=[
        jax.ShapeDtypeStruct((_N, _D), jnp.float32),
        jax.ShapeDtypeStruct((_N, _D), jnp.float32),
    ],
)


def _tc_last_body(sp_ref, h_ref, di_ref, b_ref, w_ref, b3_ref,
                  out_ref):
    di = di_ref[...]
    t = di * (sp_ref[0] + sp_ref[1]) + di * di * h_ref[...] + b_ref[...]
    t = jnp.maximum(t, 0.0)
    out_ref[...] = (
        jnp.dot(t, w_ref[...], preferred_element_type=jnp.float32)
        + b3_ref[...]
    )


_tc_last = pl.pallas_call(
    _tc_last_body,
    grid=(_G,),
    in_specs=[
        pl.BlockSpec((_NC, _R, _D), lambda i: (0, i, 0)),
        pl.BlockSpec((_R, _D), lambda i: (i, 0)),
        pl.BlockSpec((_R, 1), lambda i: (i, 0)),
        pl.BlockSpec((1, _D), lambda i: (0, 0)),
        pl.BlockSpec((_D, _NCLS), lambda i: (0, 0)),
        pl.BlockSpec((1, _NCLS), lambda i: (0, 0)),
    ],
    out_specs=pl.BlockSpec((_R, _NCLS), lambda i: (i, 0)),
    out_shape=jax.ShapeDtypeStruct((_N, _NCLS), jnp.float32),
)


def kernel(x, edge_index, W1, b1, W2, b2, W3, b3):
    src = edge_index[0]
    dst = edge_index[1]

    pad = _EPAD - _E
    pad_src = (jnp.arange(pad, dtype=src.dtype) * 97) % _N
    pad_dst = _N + (jnp.arange(pad, dtype=dst.dtype) % (_NPAD - _N))
    src3 = jnp.concatenate([src, pad_src])
    dst3 = jnp.concatenate([dst, pad_dst])

    deg_p = _deg_call(dst3.reshape(_NW, _NCH, _LC))

    h1, g1, dinv = _tc_first(x, W1, deg_p[:, :, None])

    acc1 = _gather_scatter_call(g1, src3, dst3)
    h2, g2 = _tc_mid(acc1, h1, dinv, b1.reshape(1, _D), W2)

    acc2 = _gather_scatter_call(g2, src3, dst3)
    return _tc_last(acc2, h2, dinv, b2.reshape(1, _D), W3,
                    b3.reshape(1, _NCLS))
